# baseline (speedup 1.0000x reference)
import functools

import jax
import jax.numpy as jnp
from jax import lax
from jax.experimental import pallas as pl
from jax.experimental.pallas import tpu as pltpu
from jax.experimental.pallas import tpu_sc as plsc

BATCH = 16384
EMBED = 16
N_ROWS = 1000000
REG_SCALE = 1e-05 * 0.5

_INFO = plsc.get_sparse_core_info()
NC = _INFO.num_cores
NS = _INFO.num_subcores
NW = NC * NS
BPW = BATCH // NW
GROUPS = BPW // 16

NT = 7813
TBASE = NT // NW
TEXTRA = NT - TBASE * NW
TMAX = TBASE + 1

CHUNK = 128
NCHUNK = BPW // CHUNK
CGROUPS = CHUNK // 16


def _repack_body(ue3, ie3, uc4, ic4,
                 inb_u, inb_i, outb_u, outb_i, sem_in, sem_out):
    w = lax.axis_index("s") * NC + lax.axis_index("c")
    start = w * TBASE + jnp.minimum(w, TEXTRA)
    cnt = jnp.where(w < TEXTRA, TBASE + 1, TBASE)

    step16 = lax.iota(jnp.int32, 16) * 16
    half01 = lax.iota(jnp.int32, 16) >> 3
    mod112 = step16 & 127
    cols = [mod112 + f for f in range(16)]

    def issue_in(t, slot):
        col = pl.multiple_of((start + t) * 128, 128)
        pltpu.async_copy(ue3.at[:, :, pl.ds(col, 128)],
                         inb_u.at[slot], sem_in)
        pltpu.async_copy(ie3.at[:, :, pl.ds(col, 128)],
                         inb_i.at[slot], sem_in)

    def wait_in(slot):
        pltpu.make_async_copy(ue3.at[:, :, pl.ds(0, 128)],
                              inb_u.at[slot], sem_in).wait()
        pltpu.make_async_copy(ue3.at[:, :, pl.ds(0, 128)],
                              inb_i.at[slot], sem_in).wait()

    def issue_out(t, slot):
        tt = start + t
        pltpu.async_copy(outb_u.at[slot], uc4.at[tt], sem_out)
        pltpu.async_copy(outb_i.at[slot], ic4.at[tt], sem_out)

    def wait_out(slot):
        pltpu.make_async_copy(outb_u.at[slot], uc4.at[0], sem_out).wait()
        pltpu.make_async_copy(outb_i.at[slot], ic4.at[0], sem_out).wait()

    def repack2(slot):
        for l0 in range(0, 128, 16):
            rows = half01 + (l0 >> 3)
            for t0 in range(2):
                for s in range(8):
                    f = t0 * 8 + s
                    vu = inb_u[slot, t0, s, pl.ds(l0, 16)]
                    vi = inb_i[slot, t0, s, pl.ds(l0, 16)]
                    plsc.store_scatter(outb_u.at[slot], [rows, cols[f]], vu)
                    plsc.store_scatter(outb_i.at[slot], [rows, cols[f]], vi)

    issue_in(jnp.int32(0), jnp.int32(0))

    @pl.loop(0, TMAX)
    def _(t):
        @pl.when(t < cnt)
        def _():
            slot = lax.rem(t, 2)
            wait_in(slot)

            @pl.when(t + 1 < cnt)
            def _():
                issue_in(t + 1, 1 - slot)

            @pl.when(t >= 2)
            def _():
                wait_out(slot)

            repack2(slot)
            issue_out(t, slot)

    wait_out(lax.rem(cnt, 2))
    wait_out(lax.rem(cnt + 1, 2))


@functools.partial(
    pl.kernel,
    mesh=plsc.VectorSubcoreMesh(core_axis_name="c", subcore_axis_name="s"),
    compiler_params=pltpu.CompilerParams(
        needs_layout_passes=False, use_tc_tiling_on_sc=True),
    out_type=[
        jax.ShapeDtypeStruct((NT, 16, 128), jnp.float32),
        jax.ShapeDtypeStruct((NT, 16, 128), jnp.float32),
    ],
    scratch_types=[
        pltpu.VMEM((2, 2, 8, 128), jnp.float32),
        pltpu.VMEM((2, 2, 8, 128), jnp.float32),
        pltpu.VMEM((2, 16, 128), jnp.float32),
        pltpu.VMEM((2, 16, 128), jnp.float32),
        pltpu.SemaphoreType.DMA,
        pltpu.SemaphoreType.DMA,
    ],
)
def _repack_kernel(ue3, ie3, uc4, ic4,
                   inb_u, inb_i, outb_u, outb_i, sem_in, sem_out):
    _repack_body(ue3, ie3, uc4, ic4,
                 inb_u, inb_i, outb_u, outb_i, sem_in, sem_out)


def _gather_body(user_h, pos_h, neg_h, ue_h, ie_h,
                 d_out, acc_out,
                 idx_u, idx_p, idx_n, blk_u, blk_p, blk_n,
                 bu_v, bp_v, bn_v, d_v, acc_v, sem):
    wid = lax.axis_index("s") * NC + lax.axis_index("c")
    base = wid * BPW

    pltpu.sync_copy(user_h.at[pl.ds(base, BPW)], idx_u)
    pltpu.sync_copy(pos_h.at[pl.ds(base, BPW)], idx_p)
    pltpu.sync_copy(neg_h.at[pl.ds(base, BPW)], idx_n)

    def mkblk(s, idx_ref, blk_ref):
        v = idx_ref[pl.ds(s * 16, 16)]
        blk_ref[pl.ds(s * 16, 16)] = lax.shift_right_logical(v, 3)

    @pl.loop(0, BPW // 16)
    def _(s):
        mkblk(s, idx_u, blk_u)
        mkblk(s, idx_p, blk_p)
        mkblk(s, idx_n, blk_n)

    row0 = lax.iota(jnp.int32, 16)

    def chunk_gather(c):
        cb = c * CHUNK
        cu = pltpu.async_copy(ue_h.at[blk_u.at[pl.ds(cb, CHUNK)]], bu_v, sem)
        cp = pltpu.async_copy(ie_h.at[blk_p.at[pl.ds(cb, CHUNK)]], bp_v, sem)
        cn = pltpu.async_copy(ie_h.at[blk_n.at[pl.ds(cb, CHUNK)]], bn_v, sem)
        return cu, cp, cn

    def chunk_compute(c, acc):
        cb = c * CHUNK

        def group(g, acc):
            rows = g * 16 + row0
            off_u = (idx_u[pl.ds(cb + g * 16, 16)] & 7) * 16
            off_p = (idx_p[pl.ds(cb + g * 16, 16)] & 7) * 16
            off_n = (idx_n[pl.ds(cb + g * 16, 16)] & 7) * 16
            dvec = jnp.zeros((16,), jnp.float32)
            for j in range(16):
                uc = plsc.load_gather(bu_v, [rows, off_u + j])
                pc = plsc.load_gather(bp_v, [rows, off_p + j])
                nc = plsc.load_gather(bn_v, [rows, off_n + j])
                dvec = dvec + uc * (pc - nc)
                acc = acc + uc * uc + pc * pc + nc * nc
            d_v[pl.ds(cb + g * 16, 16)] = dvec
            return acc

        return lax.fori_loop(0, CGROUPS, group, acc)

    def run_chunk(c, acc):
        cu, cp, cn = chunk_gather(c)
        cu.wait()
        cp.wait()
        cn.wait()
        return chunk_compute(c, acc)

    acc = lax.fori_loop(0, NCHUNK, run_chunk, jnp.zeros((16,), jnp.float32))
    acc_v[...] = acc

    pltpu.sync_copy(d_v, d_out.at[pl.ds(base, BPW)])
    pltpu.sync_copy(acc_v, acc_out.at[wid])


@functools.partial(
    pl.kernel,
    mesh=plsc.VectorSubcoreMesh(core_axis_name="c", subcore_axis_name="s"),
    compiler_params=pltpu.CompilerParams(
        needs_layout_passes=False, use_tc_tiling_on_sc=False),
    out_type=[
        jax.ShapeDtypeStruct((BATCH,), jnp.float32),
        jax.ShapeDtypeStruct((NW, EMBED), jnp.float32),
    ],
    scratch_types=[
        pltpu.VMEM((BPW,), jnp.int32),
        pltpu.VMEM((BPW,), jnp.int32),
        pltpu.VMEM((BPW,), jnp.int32),
        pltpu.VMEM((BPW,), jnp.int32),
        pltpu.VMEM((BPW,), jnp.int32),
        pltpu.VMEM((BPW,), jnp.int32),
        pltpu.VMEM((CHUNK, 128), jnp.float32),
        pltpu.VMEM((CHUNK, 128), jnp.float32),
        pltpu.VMEM((CHUNK, 128), jnp.float32),
        pltpu.VMEM((BPW,), jnp.float32),
        pltpu.VMEM((EMBED,), jnp.float32),
        pltpu.SemaphoreType.DMA,
    ],
)
def _gather_kernel(user_h, pos_h, neg_h, ue_h, ie_h, d_out, acc_out,
                   idx_u, idx_p, idx_n, blk_u, blk_p, blk_n,
                   bu_v, bp_v, bn_v, d_v, acc_v, sem):
    _gather_body(user_h, pos_h, neg_h, ue_h, ie_h, d_out, acc_out,
                 idx_u, idx_p, idx_n, blk_u, blk_p, blk_n,
                 bu_v, bp_v, bn_v, d_v, acc_v, sem)


def _tc_body(d_ref, acc_ref, bpr_ref, reg_ref):
    x = d_ref[...]
    s = jnp.log(jax.nn.sigmoid(x))
    bpr_ref[0, 0] = -jnp.sum(s) / jnp.float32(BATCH)
    reg_ref[0, 0] = jnp.float32(REG_SCALE) * jnp.sum(acc_ref[...])


_tc_finish = pl.pallas_call(
    _tc_body,
    out_shape=[
        jax.ShapeDtypeStruct((1, 1), jnp.float32),
        jax.ShapeDtypeStruct((1, 1), jnp.float32),
    ],
    in_specs=[
        pl.BlockSpec(memory_space=pltpu.VMEM),
        pl.BlockSpec(memory_space=pltpu.VMEM),
    ],
    out_specs=[
        pl.BlockSpec(memory_space=pltpu.SMEM),
        pl.BlockSpec(memory_space=pltpu.SMEM),
    ],
)


def kernel(user, pos, neg, user_embedding, item_embedding):
    user = user.astype(jnp.int32)
    pos = pos.astype(jnp.int32)
    neg = neg.astype(jnp.int32)
    ue3 = user_embedding.T.reshape(2, 8, N_ROWS)
    ie3 = item_embedding.T.reshape(2, 8, N_ROWS)
    uc4, ic4 = _repack_kernel(ue3, ie3)
    uc = uc4.reshape(NT * 16, 128)
    ic = ic4.reshape(NT * 16, 128)
    d, acc = _gather_kernel(user, pos, neg, uc, ic)
    bpr, reg = _tc_finish(d.reshape(128, 128), acc)
    return (bpr[0, 0], reg[0, 0])


# speedup vs baseline: 1.1651x; 1.1651x over previous
import functools

import jax
import jax.numpy as jnp
from jax import lax
from jax.experimental import pallas as pl
from jax.experimental.pallas import tpu as pltpu
from jax.experimental.pallas import tpu_sc as plsc

BATCH = 16384
EMBED = 16
N_ROWS = 1000000
REG_SCALE = 1e-05 * 0.5

_INFO = plsc.get_sparse_core_info()
NC = _INFO.num_cores
NS = _INFO.num_subcores
NW = NC * NS
BPW = BATCH // NW
GROUPS = BPW // 16

NT = 7813
TBASE = NT // NW
TEXTRA = NT - TBASE * NW
TMAX = TBASE + 1

CHUNK = 128
NCHUNK = BPW // CHUNK
CGROUPS = CHUNK // 16


def _repack_body(ue3, ie3, uc4, ic4,
                 inb_u, inb_i, outb_u, outb_i, sem_in, sem_out):
    w = lax.axis_index("s") * NC + lax.axis_index("c")
    start = w * TBASE + jnp.minimum(w, TEXTRA)
    cnt = jnp.where(w < TEXTRA, TBASE + 1, TBASE)

    step16 = lax.iota(jnp.int32, 16) * 16
    half01 = lax.iota(jnp.int32, 16) >> 3
    mod112 = step16 & 127
    cols = [mod112 + f for f in range(16)]

    def issue_in(t, slot):
        col = pl.multiple_of((start + t) * 128, 128)
        pltpu.async_copy(ue3.at[:, :, pl.ds(col, 128)],
                         inb_u.at[slot], sem_in)
        pltpu.async_copy(ie3.at[:, :, pl.ds(col, 128)],
                         inb_i.at[slot], sem_in)

    def wait_in(slot):
        pltpu.make_async_copy(ue3.at[:, :, pl.ds(0, 128)],
                              inb_u.at[slot], sem_in).wait()
        pltpu.make_async_copy(ue3.at[:, :, pl.ds(0, 128)],
                              inb_i.at[slot], sem_in).wait()

    def issue_out(t, slot):
        tt = start + t
        pltpu.async_copy(outb_u.at[slot], uc4.at[tt], sem_out)
        pltpu.async_copy(outb_i.at[slot], ic4.at[tt], sem_out)

    def wait_out(slot):
        pltpu.make_async_copy(outb_u.at[slot], uc4.at[0], sem_out).wait()
        pltpu.make_async_copy(outb_i.at[slot], ic4.at[0], sem_out).wait()

    def repack(inb, outb, slot):
        for l0 in range(0, 128, 16):
            rows = half01 + (l0 >> 3)
            for t0 in range(2):
                for s in range(8):
                    f = t0 * 8 + s
                    v = inb[slot, t0, s, pl.ds(l0, 16)]
                    plsc.store_scatter(outb.at[slot], [rows, cols[f]], v)

    issue_in(jnp.int32(0), jnp.int32(0))

    @pl.loop(0, TMAX)
    def _(t):
        @pl.when(t < cnt)
        def _():
            slot = lax.rem(t, 2)
            wait_in(slot)

            @pl.when(t + 1 < cnt)
            def _():
                issue_in(t + 1, 1 - slot)

            @pl.when(t >= 2)
            def _():
                wait_out(slot)

            repack(inb_u, outb_u, slot)
            repack(inb_i, outb_i, slot)
            issue_out(t, slot)

    wait_out(lax.rem(cnt, 2))
    wait_out(lax.rem(cnt + 1, 2))


@functools.partial(
    pl.kernel,
    mesh=plsc.VectorSubcoreMesh(core_axis_name="c", subcore_axis_name="s"),
    compiler_params=pltpu.CompilerParams(
        needs_layout_passes=False, use_tc_tiling_on_sc=True),
    out_type=[
        jax.ShapeDtypeStruct((NT, 16, 128), jnp.float32),
        jax.ShapeDtypeStruct((NT, 16, 128), jnp.float32),
    ],
    scratch_types=[
        pltpu.VMEM((2, 2, 8, 128), jnp.float32),
        pltpu.VMEM((2, 2, 8, 128), jnp.float32),
        pltpu.VMEM((2, 16, 128), jnp.float32),
        pltpu.VMEM((2, 16, 128), jnp.float32),
        pltpu.SemaphoreType.DMA,
        pltpu.SemaphoreType.DMA,
    ],
)
def _repack_kernel(ue3, ie3, uc4, ic4,
                   inb_u, inb_i, outb_u, outb_i, sem_in, sem_out):
    _repack_body(ue3, ie3, uc4, ic4,
                 inb_u, inb_i, outb_u, outb_i, sem_in, sem_out)


def _gather_body(user_h, pos_h, neg_h, ue_h, ie_h,
                 d_out, acc_out,
                 idx_u, idx_p, idx_n, blk_u, blk_p, blk_n,
                 bu_v, bp_v, bn_v, d_v, acc_v, sem):
    wid = lax.axis_index("s") * NC + lax.axis_index("c")
    base = wid * BPW

    pltpu.sync_copy(user_h.at[pl.ds(base, BPW)], idx_u)
    pltpu.sync_copy(pos_h.at[pl.ds(base, BPW)], idx_p)
    pltpu.sync_copy(neg_h.at[pl.ds(base, BPW)], idx_n)

    def mkblk(s, idx_ref, blk_ref):
        v = idx_ref[pl.ds(s * 16, 16)]
        blk_ref[pl.ds(s * 16, 16)] = lax.shift_right_logical(v, 3)

    @pl.loop(0, BPW // 16)
    def _(s):
        mkblk(s, idx_u, blk_u)
        mkblk(s, idx_p, blk_p)
        mkblk(s, idx_n, blk_n)

    row0 = lax.iota(jnp.int32, 16)

    def chunk_gather(c):
        cb = c * CHUNK
        cu = pltpu.async_copy(ue_h.at[blk_u.at[pl.ds(cb, CHUNK)]], bu_v, sem)
        cp = pltpu.async_copy(ie_h.at[blk_p.at[pl.ds(cb, CHUNK)]], bp_v, sem)
        cn = pltpu.async_copy(ie_h.at[blk_n.at[pl.ds(cb, CHUNK)]], bn_v, sem)
        return cu, cp, cn

    def chunk_compute(c, acc):
        cb = c * CHUNK

        def group(g, acc):
            rows = g * 16 + row0
            off_u = (idx_u[pl.ds(cb + g * 16, 16)] & 7) * 16
            off_p = (idx_p[pl.ds(cb + g * 16, 16)] & 7) * 16
            off_n = (idx_n[pl.ds(cb + g * 16, 16)] & 7) * 16
            dvec = jnp.zeros((16,), jnp.float32)
            for j in range(16):
                uc = plsc.load_gather(bu_v, [rows, off_u + j])
                pc = plsc.load_gather(bp_v, [rows, off_p + j])
                nc = plsc.load_gather(bn_v, [rows, off_n + j])
                dvec = dvec + uc * (pc - nc)
                acc = acc + uc * uc + pc * pc + nc * nc
            d_v[pl.ds(cb + g * 16, 16)] = dvec
            return acc

        return lax.fori_loop(0, CGROUPS, group, acc)

    def run_chunk(c, acc):
        cu, cp, cn = chunk_gather(c)
        cu.wait()
        cp.wait()
        cn.wait()
        return chunk_compute(c, acc)

    acc = lax.fori_loop(0, NCHUNK, run_chunk, jnp.zeros((16,), jnp.float32))
    acc_v[...] = acc

    pltpu.sync_copy(d_v, d_out.at[pl.ds(base, BPW)])
    pltpu.sync_copy(acc_v, acc_out.at[wid])


@functools.partial(
    pl.kernel,
    mesh=plsc.VectorSubcoreMesh(core_axis_name="c", subcore_axis_name="s"),
    compiler_params=pltpu.CompilerParams(
        needs_layout_passes=False, use_tc_tiling_on_sc=False),
    out_type=[
        jax.ShapeDtypeStruct((BATCH,), jnp.float32),
        jax.ShapeDtypeStruct((NW, EMBED), jnp.float32),
    ],
    scratch_types=[
        pltpu.VMEM((BPW,), jnp.int32),
        pltpu.VMEM((BPW,), jnp.int32),
        pltpu.VMEM((BPW,), jnp.int32),
        pltpu.VMEM((BPW,), jnp.int32),
        pltpu.VMEM((BPW,), jnp.int32),
        pltpu.VMEM((BPW,), jnp.int32),
        pltpu.VMEM((CHUNK, 128), jnp.float32),
        pltpu.VMEM((CHUNK, 128), jnp.float32),
        pltpu.VMEM((CHUNK, 128), jnp.float32),
        pltpu.VMEM((BPW,), jnp.float32),
        pltpu.VMEM((EMBED,), jnp.float32),
        pltpu.SemaphoreType.DMA,
    ],
)
def _gather_kernel(user_h, pos_h, neg_h, ue_h, ie_h, d_out, acc_out,
                   idx_u, idx_p, idx_n, blk_u, blk_p, blk_n,
                   bu_v, bp_v, bn_v, d_v, acc_v, sem):
    _gather_body(user_h, pos_h, neg_h, ue_h, ie_h, d_out, acc_out,
                 idx_u, idx_p, idx_n, blk_u, blk_p, blk_n,
                 bu_v, bp_v, bn_v, d_v, acc_v, sem)


def _tc_body(d_ref, acc_ref, bpr_ref, reg_ref):
    x = d_ref[...]
    s = jnp.log(jax.nn.sigmoid(x))
    bpr_ref[0, 0] = -jnp.sum(s) / jnp.float32(BATCH)
    reg_ref[0, 0] = jnp.float32(REG_SCALE) * jnp.sum(acc_ref[...])


_tc_finish = pl.pallas_call(
    _tc_body,
    out_shape=[
        jax.ShapeDtypeStruct((1, 1), jnp.float32),
        jax.ShapeDtypeStruct((1, 1), jnp.float32),
    ],
    in_specs=[
        pl.BlockSpec(memory_space=pltpu.VMEM),
        pl.BlockSpec(memory_space=pltpu.VMEM),
    ],
    out_specs=[
        pl.BlockSpec(memory_space=pltpu.SMEM),
        pl.BlockSpec(memory_space=pltpu.SMEM),
    ],
)


def kernel(user, pos, neg, user_embedding, item_embedding):
    user = user.astype(jnp.int32)
    pos = pos.astype(jnp.int32)
    neg = neg.astype(jnp.int32)
    ue3 = user_embedding.T.reshape(2, 8, N_ROWS)
    ie3 = item_embedding.T.reshape(2, 8, N_ROWS)
    uc4, ic4 = _repack_kernel(ue3, ie3)
    uc = uc4.reshape(NT * 16, 128)
    ic = ic4.reshape(NT * 16, 128)
    d, acc = _gather_kernel(user, pos, neg, uc, ic)
    bpr, reg = _tc_finish(d.reshape(128, 128), acc)
    return (bpr[0, 0], reg[0, 0])


# speedup vs baseline: 1.2055x; 1.0347x over previous
import functools

import jax
import jax.numpy as jnp
from jax import lax
from jax.experimental import pallas as pl
from jax.experimental.pallas import tpu as pltpu
from jax.experimental.pallas import tpu_sc as plsc

BATCH = 16384
EMBED = 16
N_ROWS = 1000000
REG_SCALE = 1e-05 * 0.5

_INFO = plsc.get_sparse_core_info()
NC = _INFO.num_cores
NS = _INFO.num_subcores
NW = NC * NS
BPW = BATCH // NW
GROUPS = BPW // 16

NT = 7813
TBASE = NT // NW
TEXTRA = NT - TBASE * NW
TMAX = TBASE + 1

CHUNK = 128
NCHUNK = BPW // CHUNK
CGROUPS = CHUNK // 16


def _repack_body(ue3, ie3, uc4, ic4,
                 inb_u, inb_i, outb_u, outb_i, sem_in, sem_out):
    w = lax.axis_index("s") * NC + lax.axis_index("c")
    start = w * TBASE + jnp.minimum(w, TEXTRA)
    cnt = jnp.where(w < TEXTRA, TBASE + 1, TBASE)

    step16 = lax.iota(jnp.int32, 16) * 16
    half01 = lax.iota(jnp.int32, 16) >> 3
    mod112 = step16 & 127
    cols = [mod112 + f for f in range(16)]

    def issue_in(t, slot):
        col = pl.multiple_of((start + t) * 128, 128)
        pltpu.async_copy(ue3.at[:, :, pl.ds(col, 128)],
                         inb_u.at[slot], sem_in)
        pltpu.async_copy(ie3.at[:, :, pl.ds(col, 128)],
                         inb_i.at[slot], sem_in)

    def wait_in(slot):
        pltpu.make_async_copy(ue3.at[:, :, pl.ds(0, 128)],
                              inb_u.at[slot], sem_in).wait()
        pltpu.make_async_copy(ue3.at[:, :, pl.ds(0, 128)],
                              inb_i.at[slot], sem_in).wait()

    def issue_out(t, slot):
        tt = start + t
        pltpu.async_copy(outb_u.at[slot], uc4.at[tt], sem_out)
        pltpu.async_copy(outb_i.at[slot], ic4.at[tt], sem_out)

    def wait_out(slot):
        pltpu.make_async_copy(outb_u.at[slot], uc4.at[0], sem_out).wait()
        pltpu.make_async_copy(outb_i.at[slot], ic4.at[0], sem_out).wait()

    def repack(inb, outb, slot):
        for l0 in range(0, 128, 16):
            rows = half01 + (l0 >> 3)
            for t0 in range(2):
                for s in range(8):
                    f = t0 * 8 + s
                    v = inb[slot, t0, s, pl.ds(l0, 16)]
                    plsc.store_scatter(outb.at[slot], [rows, cols[f]], v)

    issue_in(jnp.int32(0), jnp.int32(0))

    @pl.loop(0, TMAX, unroll=2)
    def _(t):
        @pl.when(t < cnt)
        def _():
            slot = lax.rem(t, 2)
            wait_in(slot)

            @pl.when(t + 1 < cnt)
            def _():
                issue_in(t + 1, 1 - slot)

            @pl.when(t >= 2)
            def _():
                wait_out(slot)

            repack(inb_u, outb_u, slot)
            repack(inb_i, outb_i, slot)
            issue_out(t, slot)

    wait_out(lax.rem(cnt, 2))
    wait_out(lax.rem(cnt + 1, 2))


@functools.partial(
    pl.kernel,
    mesh=plsc.VectorSubcoreMesh(core_axis_name="c", subcore_axis_name="s"),
    compiler_params=pltpu.CompilerParams(
        needs_layout_passes=False, use_tc_tiling_on_sc=True),
    out_type=[
        jax.ShapeDtypeStruct((NT, 16, 128), jnp.float32),
        jax.ShapeDtypeStruct((NT, 16, 128), jnp.float32),
    ],
    scratch_types=[
        pltpu.VMEM((2, 2, 8, 128), jnp.float32),
        pltpu.VMEM((2, 2, 8, 128), jnp.float32),
        pltpu.VMEM((2, 16, 128), jnp.float32),
        pltpu.VMEM((2, 16, 128), jnp.float32),
        pltpu.SemaphoreType.DMA,
        pltpu.SemaphoreType.DMA,
    ],
)
def _repack_kernel(ue3, ie3, uc4, ic4,
                   inb_u, inb_i, outb_u, outb_i, sem_in, sem_out):
    _repack_body(ue3, ie3, uc4, ic4,
                 inb_u, inb_i, outb_u, outb_i, sem_in, sem_out)


def _gather_body(user_h, pos_h, neg_h, ue_h, ie_h,
                 d_out, acc_out,
                 idx_u, idx_p, idx_n, blk_u, blk_p, blk_n,
                 bu_v, bp_v, bn_v, d_v, acc_v, sem):
    wid = lax.axis_index("s") * NC + lax.axis_index("c")
    base = wid * BPW

    pltpu.sync_copy(user_h.at[pl.ds(base, BPW)], idx_u)
    pltpu.sync_copy(pos_h.at[pl.ds(base, BPW)], idx_p)
    pltpu.sync_copy(neg_h.at[pl.ds(base, BPW)], idx_n)

    def mkblk(s, idx_ref, blk_ref):
        v = idx_ref[pl.ds(s * 16, 16)]
        blk_ref[pl.ds(s * 16, 16)] = lax.shift_right_logical(v, 3)

    @pl.loop(0, BPW // 16)
    def _(s):
        mkblk(s, idx_u, blk_u)
        mkblk(s, idx_p, blk_p)
        mkblk(s, idx_n, blk_n)

    row0 = lax.iota(jnp.int32, 16)

    def chunk_gather(c):
        cb = c * CHUNK
        cu = pltpu.async_copy(ue_h.at[blk_u.at[pl.ds(cb, CHUNK)]], bu_v, sem)
        cp = pltpu.async_copy(ie_h.at[blk_p.at[pl.ds(cb, CHUNK)]], bp_v, sem)
        cn = pltpu.async_copy(ie_h.at[blk_n.at[pl.ds(cb, CHUNK)]], bn_v, sem)
        return cu, cp, cn

    def chunk_compute(c, acc):
        cb = c * CHUNK

        def group(g, acc):
            rows = g * 16 + row0
            off_u = (idx_u[pl.ds(cb + g * 16, 16)] & 7) * 16
            off_p = (idx_p[pl.ds(cb + g * 16, 16)] & 7) * 16
            off_n = (idx_n[pl.ds(cb + g * 16, 16)] & 7) * 16
            dvec = jnp.zeros((16,), jnp.float32)
            for j in range(16):
                uc = plsc.load_gather(bu_v, [rows, off_u + j])
                pc = plsc.load_gather(bp_v, [rows, off_p + j])
                nc = plsc.load_gather(bn_v, [rows, off_n + j])
                dvec = dvec + uc * (pc - nc)
                acc = acc + uc * uc + pc * pc + nc * nc
            d_v[pl.ds(cb + g * 16, 16)] = dvec
            return acc

        return lax.fori_loop(0, CGROUPS, group, acc)

    def run_chunk(c, acc):
        cu, cp, cn = chunk_gather(c)
        cu.wait()
        cp.wait()
        cn.wait()
        return chunk_compute(c, acc)

    acc = lax.fori_loop(0, NCHUNK, run_chunk, jnp.zeros((16,), jnp.float32))
    acc_v[...] = acc

    pltpu.sync_copy(d_v, d_out.at[pl.ds(base, BPW)])
    pltpu.sync_copy(acc_v, acc_out.at[wid])


@functools.partial(
    pl.kernel,
    mesh=plsc.VectorSubcoreMesh(core_axis_name="c", subcore_axis_name="s"),
    compiler_params=pltpu.CompilerParams(
        needs_layout_passes=False, use_tc_tiling_on_sc=False),
    out_type=[
        jax.ShapeDtypeStruct((BATCH,), jnp.float32),
        jax.ShapeDtypeStruct((NW, EMBED), jnp.float32),
    ],
    scratch_types=[
        pltpu.VMEM((BPW,), jnp.int32),
        pltpu.VMEM((BPW,), jnp.int32),
        pltpu.VMEM((BPW,), jnp.int32),
        pltpu.VMEM((BPW,), jnp.int32),
        pltpu.VMEM((BPW,), jnp.int32),
        pltpu.VMEM((BPW,), jnp.int32),
        pltpu.VMEM((CHUNK, 128), jnp.float32),
        pltpu.VMEM((CHUNK, 128), jnp.float32),
        pltpu.VMEM((CHUNK, 128), jnp.float32),
        pltpu.VMEM((BPW,), jnp.float32),
        pltpu.VMEM((EMBED,), jnp.float32),
        pltpu.SemaphoreType.DMA,
    ],
)
def _gather_kernel(user_h, pos_h, neg_h, ue_h, ie_h, d_out, acc_out,
                   idx_u, idx_p, idx_n, blk_u, blk_p, blk_n,
                   bu_v, bp_v, bn_v, d_v, acc_v, sem):
    _gather_body(user_h, pos_h, neg_h, ue_h, ie_h, d_out, acc_out,
                 idx_u, idx_p, idx_n, blk_u, blk_p, blk_n,
                 bu_v, bp_v, bn_v, d_v, acc_v, sem)


def _tc_body(d_ref, acc_ref, bpr_ref, reg_ref):
    x = d_ref[...]
    s = jnp.log(jax.nn.sigmoid(x))
    bpr_ref[0, 0] = -jnp.sum(s) / jnp.float32(BATCH)
    reg_ref[0, 0] = jnp.float32(REG_SCALE) * jnp.sum(acc_ref[...])


_tc_finish = pl.pallas_call(
    _tc_body,
    out_shape=[
        jax.ShapeDtypeStruct((1, 1), jnp.float32),
        jax.ShapeDtypeStruct((1, 1), jnp.float32),
    ],
    in_specs=[
        pl.BlockSpec(memory_space=pltpu.VMEM),
        pl.BlockSpec(memory_space=pltpu.VMEM),
    ],
    out_specs=[
        pl.BlockSpec(memory_space=pltpu.SMEM),
        pl.BlockSpec(memory_space=pltpu.SMEM),
    ],
)


def kernel(user, pos, neg, user_embedding, item_embedding):
    user = user.astype(jnp.int32)
    pos = pos.astype(jnp.int32)
    neg = neg.astype(jnp.int32)
    ue3 = user_embedding.T.reshape(2, 8, N_ROWS)
    ie3 = item_embedding.T.reshape(2, 8, N_ROWS)
    uc4, ic4 = _repack_kernel(ue3, ie3)
    uc = uc4.reshape(NT * 16, 128)
    ic = ic4.reshape(NT * 16, 128)
    d, acc = _gather_kernel(user, pos, neg, uc, ic)
    bpr, reg = _tc_finish(d.reshape(128, 128), acc)
    return (bpr[0, 0], reg[0, 0])
